# baseline (speedup 1.0000x reference)
import jax
import jax.numpy as jnp
from jax.experimental import pallas as pl
from jax.experimental.pallas import tpu as pltpu
from jax.experimental.pallas import tpu_sc as plsc

_B = 16
_C = 64
_HW = 1024
_K = 1024
_NUMEL = _B * _C * _HW
_NTOK = _B * _HW
_GW = 128


def _vq_body(x_ref, w_ref, idx_ref, loss_ref, perp_ref, counts_ref,
             acc_ref, iota_ref, w16_ref, wn_ref):
    b = pl.program_id(0)
    X = x_ref[0]

    @pl.when(b == 0)
    def _init():
        W = w_ref[...]
        w16_ref[...] = W.astype(jnp.bfloat16) * jnp.bfloat16(2.0)
        wn_ref[...] = jnp.sum(W * W, axis=1, keepdims=True)
        iota_ref[...] = jax.lax.broadcasted_iota(
            jnp.int32, (_K, _HW), 0).astype(jnp.float32)
        counts_ref[...] = jnp.zeros_like(counts_ref)
        acc_ref[0] = 0.0

    M2 = jnp.dot(w16_ref[...], X.astype(jnp.bfloat16),
                 preferred_element_type=jnp.float32)
    xn = jnp.sum(X * X, axis=0)
    T = (xn[None, :] + wn_ref[...]) - M2
    iota_kf = iota_ref[...]
    minv = jnp.min(T, axis=0)
    idxf = jnp.min(jnp.where(T == minv[None, :], iota_kf, float(_K)), axis=0)
    idx = idxf.astype(jnp.int32)
    idx_ref[0, 0, :] = idx

    Et = (idx[:, None]
          == jax.lax.broadcasted_iota(jnp.int32, (_HW, _K), 1))
    counts_ref[...] += jnp.sum(Et.astype(jnp.float32), axis=0)
    acc_ref[0] += jnp.sum(minv)

    @pl.when(b == _B - 1)
    def _finalize():
        loss_ref[...] = jnp.full((1, 1), acc_ref[0] * (1.25 / _NUMEL),
                                 jnp.float32)
        p = counts_ref[...] * (1.0 / _NTOK)
        perp_ref[...] = jnp.full((1, 1),
                                 jnp.exp(-jnp.sum(p * jnp.log(p + 1e-10))),
                                 jnp.float32)


def _tc_stage(x3, W):
    return pl.pallas_call(
        _vq_body,
        grid=(_B,),
        in_specs=[pl.BlockSpec((1, _C, _HW), lambda b: (b, 0, 0)),
                  pl.BlockSpec((_K, _C), lambda b: (0, 0))],
        out_specs=[pl.BlockSpec((1, 1, _HW), lambda b: (b, 0, 0)),
                   pl.BlockSpec((1, 1), lambda b: (0, 0)),
                   pl.BlockSpec((1, 1), lambda b: (0, 0))],
        out_shape=[jax.ShapeDtypeStruct((_B, 1, _HW), jnp.int32),
                   jax.ShapeDtypeStruct((1, 1), jnp.float32),
                   jax.ShapeDtypeStruct((1, 1), jnp.float32)],
        scratch_shapes=[pltpu.VMEM((_K,), jnp.float32),
                        pltpu.SMEM((1,), jnp.float32),
                        pltpu.VMEM((_K, _HW), jnp.float32),
                        pltpu.VMEM((_K, _C), jnp.bfloat16),
                        pltpu.VMEM((_K, 1), jnp.float32)],
    )(x3, W)


def _sc_gather(W, idx_flat):
    mesh = plsc.VectorSubcoreMesh(core_axis_name="core",
                                  subcore_axis_name="subcore")
    w_pad = jnp.pad(W, ((0, 0), (0, 128 - _C)))

    @pl.kernel(out_type=jax.ShapeDtypeStruct((_NTOK, 128), jnp.float32),
               mesh=mesh)
    def gather_kernel(w_hbm, i_hbm, o_hbm):
        def body(i_vmem, o_vmem):
            pltpu.sync_copy(w_hbm.at[i_vmem.at[0]], o_vmem)

        pltpu.emit_pipeline(
            body,
            grid=(_NTOK // _GW,),
            in_specs=[pl.BlockSpec((1, _GW), index_map=lambda i: (0, i))],
            out_specs=[pl.BlockSpec((_GW, 128), index_map=lambda i: (i, 0))],
            core_axis_name=("core", "subcore"),
            dimension_semantics=(pltpu.PARALLEL,),
        )(i_hbm, o_hbm)

    return gather_kernel(w_pad, idx_flat)


def kernel(inputs, W):
    x3 = inputs.reshape(_B, _C, _HW)
    idx3, loss11, perp11 = _tc_stage(x3, W)
    qflat = _sc_gather(W, idx3.reshape(1, _NTOK))[:, :_C]
    quantized_out = (qflat.reshape(_B, _HW, _C)
                     .transpose(0, 2, 1).reshape(inputs.shape))
    return (loss11[0, 0], quantized_out, perp11[0, 0], idx3.reshape(-1))


# speedup vs baseline: 1.3885x; 1.3885x over previous
import jax
import jax.numpy as jnp
from jax.experimental import pallas as pl
from jax.experimental.pallas import tpu as pltpu

_B = 16
_C = 64
_HW = 1024
_K = 1024
_NUMEL = _B * _C * _HW
_NTOK = _B * _HW
_PB = 2


def _one_slab(X, W2_16, wn, iota_kf):
    M2 = jnp.dot(W2_16, X.astype(jnp.bfloat16),
                 preferred_element_type=jnp.float32)
    xn = jnp.sum(X * X, axis=0)
    T = (xn[None, :] + wn) - M2
    minv = jnp.min(T, axis=0)
    idxf = jnp.min(jnp.where(T == minv[None, :], iota_kf, float(_K)), axis=0)
    idx = idxf.astype(jnp.int32)
    E16 = jnp.where(iota_kf == idxf[None, :], 0.5, 0.0).astype(jnp.bfloat16)
    Q = jax.lax.dot_general(W2_16, E16, (((0,), (0,)), ((), ())),
                            preferred_element_type=jnp.float32)
    return idx, Q, jnp.sum(minv)


def _vq_body(x_ref, w_ref, idx_ref, q_ref, loss_ref, perp_ref, counts_ref,
             acc_ref, iota_ref, w16_ref, wn_ref):
    b = pl.program_id(0)

    @pl.when(b == 0)
    def _init():
        W = w_ref[...]
        w16_ref[...] = W.astype(jnp.bfloat16) * jnp.bfloat16(2.0)
        wn_ref[...] = jnp.sum(W * W, axis=1, keepdims=True)
        iota_ref[...] = jax.lax.broadcasted_iota(
            jnp.int32, (_K, _HW), 0).astype(jnp.float32)
        counts_ref[...] = jnp.zeros_like(counts_ref)
        acc_ref[0] = 0.0

    W2_16 = w16_ref[...]
    wn = wn_ref[...]
    iota_kf = iota_ref[...]
    acc = acc_ref[0]
    cnt = counts_ref[...]
    for s in range(_PB):
        idx, Q, msum = _one_slab(x_ref[s], W2_16, wn, iota_kf)
        idx_ref[s, 0, :] = idx
        q_ref[s] = Q
        Et = (idx[:, None]
              == jax.lax.broadcasted_iota(jnp.int32, (_HW, _K), 1))
        cnt = cnt + jnp.sum(Et.astype(jnp.float32), axis=0)
        acc = acc + msum
    counts_ref[...] = cnt
    acc_ref[0] = acc

    @pl.when(b == _B // _PB - 1)
    def _finalize():
        loss_ref[...] = jnp.full((1, 1), acc_ref[0] * (1.25 / _NUMEL),
                                 jnp.float32)
        p = counts_ref[...] * (1.0 / _NTOK)
        perp_ref[...] = jnp.full((1, 1),
                                 jnp.exp(-jnp.sum(p * jnp.log(p + 1e-10))),
                                 jnp.float32)


def kernel(inputs, W):
    x3 = inputs.reshape(_B, _C, _HW)
    idx3, q3, loss11, perp11 = pl.pallas_call(
        _vq_body,
        grid=(_B // _PB,),
        in_specs=[pl.BlockSpec((_PB, _C, _HW), lambda b: (b, 0, 0)),
                  pl.BlockSpec((_K, _C), lambda b: (0, 0))],
        out_specs=[pl.BlockSpec((_PB, 1, _HW), lambda b: (b, 0, 0)),
                   pl.BlockSpec((_PB, _C, _HW), lambda b: (b, 0, 0)),
                   pl.BlockSpec((1, 1), lambda b: (0, 0)),
                   pl.BlockSpec((1, 1), lambda b: (0, 0))],
        out_shape=[jax.ShapeDtypeStruct((_B, 1, _HW), jnp.int32),
                   jax.ShapeDtypeStruct((_B, _C, _HW), jnp.float32),
                   jax.ShapeDtypeStruct((1, 1), jnp.float32),
                   jax.ShapeDtypeStruct((1, 1), jnp.float32)],
        scratch_shapes=[pltpu.VMEM((_K,), jnp.float32),
                        pltpu.SMEM((1,), jnp.float32),
                        pltpu.VMEM((_K, _HW), jnp.float32),
                        pltpu.VMEM((_K, _C), jnp.bfloat16),
                        pltpu.VMEM((_K, 1), jnp.float32)],
    )(x3, W)
    loss = loss11[0, 0]
    perplexity = perp11[0, 0]
    quantized_out = q3.reshape(inputs.shape)
    codebook_indices = idx3.reshape(-1)
    return (loss, quantized_out, perplexity, codebook_indices)


# speedup vs baseline: 1.4025x; 1.0101x over previous
import jax
import jax.numpy as jnp
from jax.experimental import pallas as pl
from jax.experimental.pallas import tpu as pltpu

_B = 16
_C = 64
_HW = 1024
_K = 1024
_NUMEL = _B * _C * _HW
_NTOK = _B * _HW
_PB = 4


def _one_slab(X, W2_16, wn, iota_kf):
    M2 = jnp.dot(W2_16, X.astype(jnp.bfloat16),
                 preferred_element_type=jnp.float32)
    xn = jnp.sum(X * X, axis=0)
    T = (xn[None, :] + wn) - M2
    minv = jnp.min(T, axis=0)
    idxf = jnp.min(jnp.where(T == minv[None, :], iota_kf, float(_K)), axis=0)
    idx = idxf.astype(jnp.int32)
    E16 = jnp.where(iota_kf == idxf[None, :], 0.5, 0.0).astype(jnp.bfloat16)
    Q = jax.lax.dot_general(W2_16, E16, (((0,), (0,)), ((), ())),
                            preferred_element_type=jnp.float32)
    Et = (idx[:, None]
          == jax.lax.broadcasted_iota(jnp.int32, (_HW, _K), 1))
    cnt = jnp.sum(Et.astype(jnp.float32), axis=0)
    return idx, Q, cnt, jnp.sum(minv)


def _vq_body(x_ref, w_ref, idx_ref, q_ref, loss_ref, perp_ref, counts_ref,
             acc_ref, iota_ref, w16_ref, wn_ref):
    b = pl.program_id(0)

    @pl.when(b == 0)
    def _init():
        W = w_ref[...]
        w16_ref[...] = W.astype(jnp.bfloat16) * jnp.bfloat16(2.0)
        wn_ref[...] = jnp.sum(W * W, axis=1, keepdims=True)
        iota_ref[...] = jax.lax.broadcasted_iota(
            jnp.int32, (_K, _HW), 0).astype(jnp.float32)
        counts_ref[...] = jnp.zeros_like(counts_ref)
        acc_ref[0] = 0.0

    W2_16 = w16_ref[...]
    wn = wn_ref[...]
    iota_kf = iota_ref[...]
    acc = acc_ref[0]
    cnt = counts_ref[...]
    for s in range(_PB):
        idx, Q, cnt_s, msum = _one_slab(x_ref[s], W2_16, wn, iota_kf)
        idx_ref[s, 0, :] = idx
        q_ref[s] = Q
        cnt = cnt + cnt_s
        acc = acc + msum
    counts_ref[...] = cnt
    acc_ref[0] = acc

    @pl.when(b == _B // _PB - 1)
    def _finalize():
        loss_ref[...] = jnp.full((1, 1), acc_ref[0] * (1.25 / _NUMEL),
                                 jnp.float32)
        p = counts_ref[...] * (1.0 / _NTOK)
        perp_ref[...] = jnp.full((1, 1),
                                 jnp.exp(-jnp.sum(p * jnp.log(p + 1e-10))),
                                 jnp.float32)


def kernel(inputs, W):
    x3 = inputs.reshape(_B, _C, _HW)
    idx3, q3, loss11, perp11 = pl.pallas_call(
        _vq_body,
        grid=(_B // _PB,),
        in_specs=[pl.BlockSpec((_PB, _C, _HW), lambda b: (b, 0, 0)),
                  pl.BlockSpec((_K, _C), lambda b: (0, 0))],
        out_specs=[pl.BlockSpec((_PB, 1, _HW), lambda b: (b, 0, 0)),
                   pl.BlockSpec((_PB, _C, _HW), lambda b: (b, 0, 0)),
                   pl.BlockSpec((1, 1), lambda b: (0, 0)),
                   pl.BlockSpec((1, 1), lambda b: (0, 0))],
        out_shape=[jax.ShapeDtypeStruct((_B, 1, _HW), jnp.int32),
                   jax.ShapeDtypeStruct((_B, _C, _HW), jnp.float32),
                   jax.ShapeDtypeStruct((1, 1), jnp.float32),
                   jax.ShapeDtypeStruct((1, 1), jnp.float32)],
        scratch_shapes=[pltpu.VMEM((_K,), jnp.float32),
                        pltpu.SMEM((1,), jnp.float32),
                        pltpu.VMEM((_K, _HW), jnp.float32),
                        pltpu.VMEM((_K, _C), jnp.bfloat16),
                        pltpu.VMEM((_K, 1), jnp.float32)],
    )(x3, W)
    loss = loss11[0, 0]
    perplexity = perp11[0, 0]
    quantized_out = q3.reshape(inputs.shape)
    codebook_indices = idx3.reshape(-1)
    return (loss, quantized_out, perplexity, codebook_indices)
